# Initial kernel scaffold; baseline (speedup 1.0000x reference)
import jax
import jax.numpy as jnp
from jax.experimental import pallas as pl


def kernel(img, mesh, focal, princpt, bary_coords_uv, face, pix_to_face_xy, pix_to_face_uv):
    raise NotImplementedError("write your pallas kernel here")


# speedup vs baseline: 28.2826x; 28.2826x over previous
import functools

import jax
import jax.numpy as jnp
from jax import lax
from jax.experimental import pallas as pl
from jax.experimental.pallas import tpu as pltpu
from jax.experimental.pallas import tpu_sc as plsc

B, C, H, W = 4, 3, 512, 512
V, NF = 12288, 24576
UH, UW = 512, 512
HW = H * W
UHW = UH * UW
L = 16
NW = 32
TEC_PER_B = NW // B
PIX_PER_TEC = UHW // TEC_PER_B
CHUNK = 128
NCHUNK = PIX_PER_TEC // CHUNK
MASK_HI = -65536


def _body(imgA, imgB, mesh_f, prm_in, bary_f, face_t, p2f_xy, p2f_uv, out,
          imA, imB, fash, mksh,
          mx, my, ibuf, idxb, onesb, zerob, fb, fb2, stb,
          fwb, mkib, f1i, f2i, v0b, v1b, v2b,
          i00, i01, i10, i11,
          dA00, dA01, dA10, dA11, dB00, dB01, dB10, dB11, sem):
    cid = lax.axis_index("c")
    sid = lax.axis_index("s")
    wid = cid * 16 + sid
    b = wid // TEC_PER_B
    bl = b % 2
    sub = wid % TEC_PER_B
    zeros16 = jnp.zeros((L,), jnp.int32)
    ones16i = jnp.full((L,), 1, jnp.int32)

    words = 2 * HW // L

    def st_img(i, carry):
        so = cid * 2 * HW + sid * words + i * 1024
        do = sid * words + i * 1024
        pltpu.sync_copy(imgA.at[pl.ds(so, 1024)], stb)
        pltpu.sync_copy(stb, imA.at[pl.ds(do, 1024)])
        pltpu.sync_copy(imgB.at[pl.ds(so, 1024)], stb)
        pltpu.sync_copy(stb, imB.at[pl.ds(do, 1024)])
        return carry
    lax.fori_loop(0, words // 1024, st_img, 0)

    fwords = 3 * NF // L

    def st_face(i, carry):
        pltpu.sync_copy(face_t.at[pl.ds(sid * fwords + i * 512, 512)],
                        stb.at[pl.ds(0, 512)])
        pltpu.sync_copy(stb.at[pl.ds(0, 512)],
                        fash.at[pl.ds(sid * fwords + i * 512, 512)])
        return carry
    lax.fori_loop(0, fwords // 512, st_face, 0)

    def init_cb(i, carry):
        zerob[pl.ds(i * L, L)] = zeros16
        onesb[pl.ds(i * L, L)] = ones16i
        return carry
    lax.fori_loop(0, 1024 // L, init_cb, 0)
    mrows = 2 * NF // L
    for k in range(mrows // 1024):
        pltpu.sync_copy(zerob, mksh.at[pl.ds(sid * mrows + k * 1024, 1024)])

    pltpu.sync_copy(prm_in.at[pl.ds(b * 4 * L, 4 * L)],
                    fb2.at[pl.ds(1152, 4 * L)])
    fx = fb2[pl.ds(1152, L)]
    fy = fb2[pl.ds(1152 + L, L)]
    cx = fb2[pl.ds(1152 + 2 * L, L)]
    cy = fb2[pl.ds(1152 + 3 * L, L)]

    plsc.subcore_barrier()

    def ph_b(i, carry):
        pltpu.sync_copy(p2f_xy.at[pl.ds(b * HW + sub * (HW // TEC_PER_B)
                                        + i * 1024, 1024)], ibuf)
        for g in range(1024 // L):
            v = ibuf[pl.ds(g * L, L)]
            idx = jnp.where(v < 0, NF - 1, v - b * NF) + bl * NF
            idxb[pl.ds(g * L, L)] = idx
        pltpu.sync_copy(onesb, mksh.at[idxb], add=True)
        return carry
    lax.fori_loop(0, HW // TEC_PER_B // 1024, ph_b, 0)

    def ph_a(i, carry):
        mo = b * 3 * V + i * 256
        pltpu.sync_copy(mesh_f.at[pl.ds(mo, 256)], fb2.at[pl.ds(0, 256)])
        pltpu.sync_copy(mesh_f.at[pl.ds(mo + V, 256)], fb2.at[pl.ds(256, 256)])
        pltpu.sync_copy(mesh_f.at[pl.ds(mo + 2 * V, 256)],
                        fb2.at[pl.ds(512, 256)])
        for g in range(256 // L):
            vx = fb2[pl.ds(g * L, L)]
            vy = fb2[pl.ds(256 + g * L, L)]
            vz = fb2[pl.ds(512 + g * L, L)]
            mx[pl.ds(i * 256 + g * L, L)] = vx / vz * fx + cx
            my[pl.ds(i * 256 + g * L, L)] = vy / vz * fy + cy
        return carry
    lax.fori_loop(0, V // 256, ph_a, 0)

    plsc.subcore_barrier()

    def ph_c(i, carry):
        base = sub * PIX_PER_TEC + i * CHUNK
        pltpu.sync_copy(p2f_uv.at[pl.ds(base, CHUNK)], ibuf.at[pl.ds(0, CHUNK)])
        pltpu.sync_copy(bary_f.at[pl.ds(base, CHUNK)], fb.at[pl.ds(0, CHUNK)])
        pltpu.sync_copy(bary_f.at[pl.ds(UHW + base, CHUNK)],
                        fb.at[pl.ds(CHUNK, CHUNK)])
        pltpu.sync_copy(bary_f.at[pl.ds(2 * UHW + base, CHUNK)],
                        fb.at[pl.ds(2 * CHUNK, CHUNK)])
        for g in range(CHUNK // L):
            f = ibuf[pl.ds(g * L, L)]
            fw = jnp.where(f < 0, f + NF, f)
            fwb[pl.ds(g * L, L)] = fw
            f1i[pl.ds(g * L, L)] = fw + NF
            f2i[pl.ds(g * L, L)] = fw + 2 * NF
            mkib[pl.ds(g * L, L)] = fw + bl * NF
        c0_ = pltpu.async_copy(fash.at[fwb], v0b, sem)
        c1_ = pltpu.async_copy(fash.at[f1i], v1b, sem)
        c2_ = pltpu.async_copy(fash.at[f2i], v2b, sem)
        ck_ = pltpu.async_copy(mksh.at[mkib], ibuf.at[pl.ds(CHUNK, CHUNK)],
                               sem)
        c0_.wait()
        c1_.wait()
        c2_.wait()
        ck_.wait()
        rowbase = bl * HW
        for g in range(CHUNK // L):
            f = ibuf[pl.ds(g * L, L)]
            mv = ibuf[pl.ds(CHUNK + g * L, L)]
            live = (mv != 0) & (f >= 0)
            v0 = v0b[pl.ds(g * L, L)]
            v1 = v1b[pl.ds(g * L, L)]
            v2 = v2b[pl.ds(g * L, L)]
            w0 = fb[pl.ds(g * L, L)]
            w1 = fb[pl.ds(CHUNK + g * L, L)]
            w2 = fb[pl.ds(2 * CHUNK + g * L, L)]
            xs = (w0 * plsc.load_gather(mx, [v0])
                  + w1 * plsc.load_gather(mx, [v1])
                  + w2 * plsc.load_gather(mx, [v2]))
            ys = (w0 * plsc.load_gather(my, [v0])
                  + w1 * plsc.load_gather(my, [v1])
                  + w2 * plsc.load_gather(my, [v2]))
            xt = xs.astype(jnp.int32)
            x0 = jnp.where(xt.astype(jnp.float32) > xs, xt - 1, xt)
            yt = ys.astype(jnp.int32)
            y0 = jnp.where(yt.astype(jnp.float32) > ys, yt - 1, yt)
            wx1 = xs - x0.astype(jnp.float32)
            wy1 = ys - y0.astype(jnp.float32)
            wx0 = 1.0 - wx1
            wy0 = 1.0 - wy1
            x1 = x0 + 1
            y1 = y0 + 1
            vx0 = (x0 >= 0) & (x0 <= W - 1)
            vx1 = (x1 >= 0) & (x1 <= W - 1)
            vy0 = (y0 >= 0) & (y0 <= H - 1)
            vy1 = (y1 >= 0) & (y1 <= H - 1)
            xc0 = jnp.clip(x0, 0, W - 1)
            xc1 = jnp.clip(x1, 0, W - 1)
            yc0 = jnp.clip(y0, 0, H - 1)
            yc1 = jnp.clip(y1, 0, H - 1)
            r0 = rowbase + yc0 * W
            r1 = rowbase + yc1 * W
            i00[pl.ds(g * L, L)] = r0 + xc0
            i01[pl.ds(g * L, L)] = r0 + xc1
            i10[pl.ds(g * L, L)] = r1 + xc0
            i11[pl.ds(g * L, L)] = r1 + xc1
            fb[pl.ds(3 * CHUNK + g * L, L)] = jnp.where(vx0 & vy0,
                                                        wx0 * wy0, 0.0)
            fb[pl.ds(4 * CHUNK + g * L, L)] = jnp.where(vx1 & vy0,
                                                        wx1 * wy0, 0.0)
            fb[pl.ds(5 * CHUNK + g * L, L)] = jnp.where(vx0 & vy1,
                                                        wx0 * wy1, 0.0)
            fb[pl.ds(6 * CHUNK + g * L, L)] = jnp.where(vx1 & vy1,
                                                        wx1 * wy1, 0.0)
            fb[pl.ds(7 * CHUNK + g * L, L)] = jnp.where(live, 1.0, 0.0)
        a00 = pltpu.async_copy(imA.at[i00], dA00, sem)
        a01 = pltpu.async_copy(imA.at[i01], dA01, sem)
        a10 = pltpu.async_copy(imA.at[i10], dA10, sem)
        a11 = pltpu.async_copy(imA.at[i11], dA11, sem)
        b00 = pltpu.async_copy(imB.at[i00], dB00, sem)
        b01 = pltpu.async_copy(imB.at[i01], dB01, sem)
        b10 = pltpu.async_copy(imB.at[i10], dB10, sem)
        b11 = pltpu.async_copy(imB.at[i11], dB11, sem)
        a00.wait()
        a01.wait()
        a10.wait()
        a11.wait()
        b00.wait()
        b01.wait()
        b10.wait()
        b11.wait()
        for g in range(CHUNK // L):
            w00 = fb[pl.ds(3 * CHUNK + g * L, L)]
            w01 = fb[pl.ds(4 * CHUNK + g * L, L)]
            w10 = fb[pl.ds(5 * CHUNK + g * L, L)]
            w11 = fb[pl.ds(6 * CHUNK + g * L, L)]
            lv = fb[pl.ds(7 * CHUNK + g * L, L)]
            acc0 = jnp.zeros((L,), jnp.float32)
            acc1 = jnp.zeros((L,), jnp.float32)
            acc2 = jnp.zeros((L,), jnp.float32)
            for dA, dB, wv in ((dA00, dB00, w00), (dA01, dB01, w01),
                               (dA10, dB10, w10), (dA11, dB11, w11)):
                gA = dA[pl.ds(g * L, L)]
                gB = dB[pl.ds(g * L, L)]
                c0v = lax.bitcast_convert_type(lax.shift_left(gA, 16),
                                               jnp.float32)
                c1v = lax.bitcast_convert_type(gA & MASK_HI, jnp.float32)
                c2v = lax.bitcast_convert_type(lax.shift_left(gB, 16),
                                               jnp.float32)
                acc0 = acc0 + wv * c0v
                acc1 = acc1 + wv * c1v
                acc2 = acc2 + wv * c2v
            fb2[pl.ds(768 + g * L, L)] = jnp.where(lv > 0, acc0, -1.0)
            fb2[pl.ds(896 + g * L, L)] = jnp.where(lv > 0, acc1, -1.0)
            fb2[pl.ds(1024 + g * L, L)] = jnp.where(lv > 0, acc2, -1.0)
        oo = b * 3 * UHW + base
        pltpu.sync_copy(fb2.at[pl.ds(768, CHUNK)], out.at[pl.ds(oo, CHUNK)])
        pltpu.sync_copy(fb2.at[pl.ds(896, CHUNK)],
                        out.at[pl.ds(oo + UHW, CHUNK)])
        pltpu.sync_copy(fb2.at[pl.ds(1024, CHUNK)],
                        out.at[pl.ds(oo + 2 * UHW, CHUNK)])
        return carry
    lax.fori_loop(0, NCHUNK, ph_c, 0)


_SCRATCH = [
        pltpu.VMEM_SHARED((2 * HW,), jnp.int32),
        pltpu.VMEM_SHARED((2 * HW,), jnp.int32),
        pltpu.VMEM_SHARED((3 * NF,), jnp.int32),
        pltpu.VMEM_SHARED((2 * NF,), jnp.int32),
        pltpu.VMEM((V,), jnp.float32),
        pltpu.VMEM((V,), jnp.float32),
        pltpu.VMEM((1024,), jnp.int32),
        pltpu.VMEM((1024,), jnp.int32),
        pltpu.VMEM((1024,), jnp.int32),
        pltpu.VMEM((1024,), jnp.int32),
        pltpu.VMEM((1024,), jnp.float32),
        pltpu.VMEM((2048,), jnp.float32),
        pltpu.VMEM((1024,), jnp.int32),
        pltpu.VMEM((CHUNK,), jnp.int32),
        pltpu.VMEM((CHUNK,), jnp.int32),
        pltpu.VMEM((CHUNK,), jnp.int32),
        pltpu.VMEM((CHUNK,), jnp.int32),
        pltpu.VMEM((CHUNK,), jnp.int32),
        pltpu.VMEM((CHUNK,), jnp.int32),
        pltpu.VMEM((CHUNK,), jnp.int32),
        pltpu.VMEM((CHUNK,), jnp.int32),
        pltpu.VMEM((CHUNK,), jnp.int32),
        pltpu.VMEM((CHUNK,), jnp.int32),
        pltpu.VMEM((CHUNK,), jnp.int32),
        pltpu.VMEM((CHUNK,), jnp.int32),
        pltpu.VMEM((CHUNK,), jnp.int32),
        pltpu.VMEM((CHUNK,), jnp.int32),
        pltpu.VMEM((CHUNK,), jnp.int32),
        pltpu.VMEM((CHUNK,), jnp.int32),
        pltpu.VMEM((CHUNK,), jnp.int32),
        pltpu.VMEM((CHUNK,), jnp.int32),
        pltpu.VMEM((CHUNK,), jnp.int32),
        pltpu.SemaphoreType.DMA,
]

_uv_call = functools.partial(
    pl.kernel,
    out_type=jax.ShapeDtypeStruct((B * C * UHW,), jnp.float32),
    mesh=plsc.VectorSubcoreMesh(core_axis_name="c", subcore_axis_name="s"),
    compiler_params=pltpu.CompilerParams(needs_layout_passes=False,
                                         use_tc_tiling_on_sc=False),
    scratch_types=_SCRATCH,
)(_body)


def kernel(img, mesh, focal, princpt, bary_coords_uv, face, pix_to_face_xy,
           pix_to_face_uv):
    u = lax.bitcast_convert_type(img.astype(jnp.bfloat16),
                                 jnp.uint16).astype(jnp.uint32)
    imgA = lax.bitcast_convert_type(u[:, 0] | (u[:, 1] << 16),
                                    jnp.int32).reshape(B * HW)
    imgB = lax.bitcast_convert_type(u[:, 2], jnp.int32).reshape(B * HW)
    mu = lax.bitcast_convert_type(mesh, jnp.uint32)
    mu = (mu + 0x7FFF + ((mu >> 16) & 1)) & jnp.uint32(0xFFFF0000)
    mesh_b = lax.bitcast_convert_type(mu, jnp.float32)
    mesh_f = jnp.transpose(mesh_b, (0, 2, 1)).reshape(B * 3 * V)
    prm = jnp.stack([focal[:, 0], focal[:, 1], princpt[:, 0], princpt[:, 1]],
                    axis=1)
    prm = jnp.broadcast_to(prm[:, :, None], (B, 4, L)).reshape(B * 4 * L)
    prm = prm.astype(jnp.float32)
    bary_f = jnp.transpose(bary_coords_uv.reshape(UHW, 3), (1, 0)).reshape(-1)
    face_t = jnp.transpose(face, (1, 0)).reshape(3 * NF)
    p2f_xy = pix_to_face_xy.reshape(B * HW)
    p2f_uv = pix_to_face_uv.reshape(UHW)
    out = _uv_call(imgA, imgB, mesh_f, prm, bary_f, face_t, p2f_xy, p2f_uv)
    return out.reshape(B, C, UH, UW)
